# Initial kernel scaffold; baseline (speedup 1.0000x reference)
import jax
import jax.numpy as jnp
from jax.experimental import pallas as pl


def kernel(x, table, W, b):
    raise NotImplementedError("write your pallas kernel here")


# speedup vs baseline: 2.1269x; 2.1269x over previous
import functools

import jax
import jax.numpy as jnp
from jax import lax
from jax.experimental import pallas as pl
from jax.experimental.pallas import tpu as pltpu
from jax.experimental.pallas import tpu_sc as plsc

_NUM_CORES = 2
_NUM_SUBCORES = 16
_NUM_WORKERS = _NUM_CORES * _NUM_SUBCORES
_LANES = 16


def _tc_matvec_body(tbl_ref, w_ref, b_ref, o_ref):
    o_ref[...] = (
        jnp.dot(tbl_ref[...], w_ref[...], preferred_element_type=jnp.float32)
        + b_ref[0, 0]
    )


def _tc_matvec(table, W, b, seq):
    vocab, emb = table.shape
    g = 8
    rows = vocab // g
    blk = 1000
    grid = rows // blk
    tv = table.reshape(rows, g * emb)
    w_scaled = (W[0].astype(jnp.float32) / seq)
    w_block = (
        jnp.eye(g, dtype=jnp.float32)[:, None, :] * w_scaled[None, :, None]
    ).reshape(g * emb, g)
    b_scaled = jnp.reshape(b.astype(jnp.float32), (1, 1)) / seq
    t8 = pl.pallas_call(
        _tc_matvec_body,
        grid=(grid,),
        in_specs=[
            pl.BlockSpec((blk, g * emb), lambda i: (i, 0)),
            pl.BlockSpec((g * emb, g), lambda i: (0, 0)),
            pl.BlockSpec(memory_space=pltpu.SMEM),
        ],
        out_specs=pl.BlockSpec((blk, g), lambda i: (i, 0)),
        out_shape=jax.ShapeDtypeStruct((rows, g), jnp.float32),
    )(tv, w_block, b_scaled)
    return t8.reshape(vocab)


def _sc_gather_sum(x, t):
    seq, batch = x.shape
    bpw = batch // _NUM_WORKERS
    rows = 4
    nch = seq // rows
    nseg = bpw // 128

    mesh = plsc.VectorSubcoreMesh(
        core_axis_name="c", subcore_axis_name="s",
        num_cores=_NUM_CORES, num_subcores=_NUM_SUBCORES,
    )

    @functools.partial(
        pl.kernel,
        out_type=jax.ShapeDtypeStruct((batch,), jnp.float32),
        mesh=mesh,
        scratch_types=[
            pltpu.VMEM((rows, bpw), jnp.int32),
            pltpu.VMEM((rows, bpw), jnp.float32),
            pltpu.VMEM((bpw,), jnp.float32),
            pltpu.SemaphoreType.DMA,
        ],
    )
    def sc_kernel(x_hbm, t_hbm, out_hbm, idx_v, vals_v, acc_v, sem):
        wid = lax.axis_index("s") * _NUM_CORES + lax.axis_index("c")
        base = wid * bpw

        zero = jnp.zeros((_LANES,), jnp.float32)
        for c in range(bpw // _LANES):
            acc_v[pl.ds(c * _LANES, _LANES)] = zero

        @pl.loop(0, nch)
        def _chunk(i):
            s0 = i * rows
            pltpu.sync_copy(
                x_hbm.at[pl.ds(s0, rows), pl.ds(base, bpw)], idx_v
            )
            copies = []
            for r in range(rows):
                for c in range(nseg):
                    copies.append(pltpu.async_copy(
                        t_hbm.at[idx_v.at[r, pl.ds(c * 128, 128)]],
                        vals_v.at[r, pl.ds(c * 128, 128)],
                        sem,
                    ))
            for cp in copies:
                cp.wait()
            for c in range(bpw // _LANES):
                v = acc_v[pl.ds(c * _LANES, _LANES)]
                for r in range(rows):
                    v = v + vals_v[r, pl.ds(c * _LANES, _LANES)]
                acc_v[pl.ds(c * _LANES, _LANES)] = v

        pltpu.sync_copy(acc_v, out_hbm.at[pl.ds(base, bpw)])

    return sc_kernel(x, t)


def kernel(x, table, W, b):
    seq, _ = x.shape
    t = _tc_matvec(table, W, b, seq)
    return _sc_gather_sum(x, t)


# speedup vs baseline: 2.3765x; 1.1173x over previous
import functools

import jax
import jax.numpy as jnp
from jax import lax
from jax.experimental import pallas as pl
from jax.experimental.pallas import tpu as pltpu
from jax.experimental.pallas import tpu_sc as plsc

_NUM_CORES = 2
_NUM_SUBCORES = 16
_NUM_WORKERS = _NUM_CORES * _NUM_SUBCORES
_LANES = 16


def _tc_matvec_body(tbl_ref, w_ref, b_ref, o_ref):
    o_ref[...] = (
        jnp.dot(tbl_ref[...], w_ref[...], preferred_element_type=jnp.float32)
        + b_ref[0, 0]
    )


def _tc_matvec(table, W, b, seq):
    vocab, emb = table.shape
    g = 64
    rows = vocab // g
    blk = 512
    grid = (rows + blk - 1) // blk
    tv = table.reshape(rows, g * emb)
    w_scaled = (W[0].astype(jnp.float32) / seq)
    w_block = (
        jnp.eye(g, dtype=jnp.float32)[:, None, :] * w_scaled[None, :, None]
    ).reshape(g * emb, g)
    b_scaled = jnp.reshape(b.astype(jnp.float32), (1, 1)) / seq
    t2 = pl.pallas_call(
        _tc_matvec_body,
        grid=(grid,),
        in_specs=[
            pl.BlockSpec((blk, g * emb), lambda i: (i, 0)),
            pl.BlockSpec((g * emb, g), lambda i: (0, 0)),
            pl.BlockSpec(memory_space=pltpu.SMEM),
        ],
        out_specs=pl.BlockSpec((blk, g), lambda i: (i, 0)),
        out_shape=jax.ShapeDtypeStruct((rows, g), jnp.float32),
    )(tv, w_block, b_scaled)
    return t2.reshape(vocab)


def _sc_gather_sum(x, t):
    seq, batch = x.shape
    bpw = batch // _NUM_WORKERS
    rows = 4
    nch = seq // rows
    nseg = bpw // 128

    mesh = plsc.VectorSubcoreMesh(
        core_axis_name="c", subcore_axis_name="s",
        num_cores=_NUM_CORES, num_subcores=_NUM_SUBCORES,
    )

    @functools.partial(
        pl.kernel,
        out_type=jax.ShapeDtypeStruct((batch,), jnp.float32),
        mesh=mesh,
        scratch_types=[
            pltpu.VMEM((rows, bpw), jnp.int32),
            pltpu.VMEM((rows, bpw), jnp.float32),
            pltpu.VMEM((bpw,), jnp.float32),
            pltpu.SemaphoreType.DMA,
        ],
    )
    def sc_kernel(x_hbm, t_hbm, out_hbm, idx_v, vals_v, acc_v, sem):
        wid = lax.axis_index("s") * _NUM_CORES + lax.axis_index("c")
        base = wid * bpw

        zero = jnp.zeros((_LANES,), jnp.float32)
        for c in range(bpw // _LANES):
            acc_v[pl.ds(c * _LANES, _LANES)] = zero

        @pl.loop(0, nch)
        def _chunk(i):
            s0 = i * rows
            pltpu.sync_copy(
                x_hbm.at[pl.ds(s0, rows), pl.ds(base, bpw)], idx_v
            )
            copies = []
            for r in range(rows):
                for c in range(nseg):
                    copies.append(pltpu.async_copy(
                        t_hbm.at[idx_v.at[r, pl.ds(c * 128, 128)]],
                        vals_v.at[r, pl.ds(c * 128, 128)],
                        sem,
                    ))
            for cp in copies:
                cp.wait()
            for c in range(bpw // _LANES):
                v = acc_v[pl.ds(c * _LANES, _LANES)]
                for r in range(rows):
                    v = v + vals_v[r, pl.ds(c * _LANES, _LANES)]
                acc_v[pl.ds(c * _LANES, _LANES)] = v

        pltpu.sync_copy(acc_v, out_hbm.at[pl.ds(base, bpw)])

    return sc_kernel(x, t)


def kernel(x, table, W, b):
    seq, _ = x.shape
    t = _tc_matvec(table, W, b, seq)
    return _sc_gather_sum(x, t)


# speedup vs baseline: 2.3822x; 1.0024x over previous
import functools

import jax
import jax.numpy as jnp
from jax import lax
from jax.experimental import pallas as pl
from jax.experimental.pallas import tpu as pltpu
from jax.experimental.pallas import tpu_sc as plsc

_NUM_CORES = 2
_NUM_SUBCORES = 16
_NUM_WORKERS = _NUM_CORES * _NUM_SUBCORES
_LANES = 16


def _tc_matvec_body(tbl_ref, w_ref, b_ref, o_ref):
    r = jnp.dot(tbl_ref[...], w_ref[...], preferred_element_type=jnp.float32)
    o_ref[:, 0:64] = r + b_ref[0, 0]
    o_ref[:, 64:128] = jnp.zeros_like(r)


def _tc_matvec(table, W, b, seq):
    vocab, emb = table.shape
    g = 64
    rows = vocab // g
    blk = 512
    grid = (rows + blk - 1) // blk
    tv = table.reshape(rows, g * emb)
    w_scaled = (W[0].astype(jnp.float32) / seq)
    w_block = (
        jnp.eye(g, dtype=jnp.float32)[:, None, :] * w_scaled[None, :, None]
    ).reshape(g * emb, g)
    b_scaled = jnp.reshape(b.astype(jnp.float32), (1, 1)) / seq
    t2 = pl.pallas_call(
        _tc_matvec_body,
        grid=(grid,),
        in_specs=[
            pl.BlockSpec((blk, g * emb), lambda i: (i, 0)),
            pl.BlockSpec((g * emb, g), lambda i: (0, 0)),
            pl.BlockSpec(memory_space=pltpu.SMEM),
        ],
        out_specs=pl.BlockSpec((blk, 2 * g), lambda i: (i, 0)),
        out_shape=jax.ShapeDtypeStruct((rows, 2 * g), jnp.float32),
    )(tv, w_block, b_scaled)
    return t2.reshape(rows * 2 * g)


def _sc_gather_sum(x, t):
    seq, batch = x.shape
    bpw = batch // _NUM_WORKERS
    rows = 4
    nch = seq // rows
    nseg = bpw // 128

    mesh = plsc.VectorSubcoreMesh(
        core_axis_name="c", subcore_axis_name="s",
        num_cores=_NUM_CORES, num_subcores=_NUM_SUBCORES,
    )

    @functools.partial(
        pl.kernel,
        out_type=jax.ShapeDtypeStruct((batch,), jnp.float32),
        mesh=mesh,
        scratch_types=[
            pltpu.VMEM((rows, bpw), jnp.int32),
            pltpu.VMEM((rows, bpw), jnp.float32),
            pltpu.VMEM((bpw,), jnp.float32),
            pltpu.SemaphoreType.DMA,
        ],
    )
    def sc_kernel(x_hbm, t_hbm, out_hbm, idx_v, vals_v, acc_v, sem):
        wid = lax.axis_index("s") * _NUM_CORES + lax.axis_index("c")
        base = wid * bpw

        zero = jnp.zeros((_LANES,), jnp.float32)
        for c in range(bpw // _LANES):
            acc_v[pl.ds(c * _LANES, _LANES)] = zero

        @pl.loop(0, nch)
        def _chunk(i):
            s0 = i * rows
            pltpu.sync_copy(
                x_hbm.at[pl.ds(s0, rows), pl.ds(base, bpw)], idx_v
            )
            for r in range(rows):
                for c in range(bpw // _LANES):
                    v = idx_v[r, pl.ds(c * _LANES, _LANES)]
                    idx_v[r, pl.ds(c * _LANES, _LANES)] = v + (v & -64)
            copies = []
            for r in range(rows):
                for c in range(nseg):
                    copies.append(pltpu.async_copy(
                        t_hbm.at[idx_v.at[r, pl.ds(c * 128, 128)]],
                        vals_v.at[r, pl.ds(c * 128, 128)],
                        sem,
                    ))
            for cp in copies:
                cp.wait()
            for c in range(bpw // _LANES):
                v = acc_v[pl.ds(c * _LANES, _LANES)]
                for r in range(rows):
                    v = v + vals_v[r, pl.ds(c * _LANES, _LANES)]
                acc_v[pl.ds(c * _LANES, _LANES)] = v

        pltpu.sync_copy(acc_v, out_hbm.at[pl.ds(base, bpw)])

    return sc_kernel(x, t)


def kernel(x, table, W, b):
    seq, _ = x.shape
    t = _tc_matvec(table, W, b, seq)
    return _sc_gather_sum(x, t)


# speedup vs baseline: 6.3368x; 2.6600x over previous
import functools

import jax
import jax.numpy as jnp
from jax import lax
from jax.experimental import pallas as pl
from jax.experimental.pallas import tpu as pltpu
from jax.experimental.pallas import tpu_sc as plsc

_NUM_CORES = 2
_NUM_SUBCORES = 16
_NUM_WORKERS = _NUM_CORES * _NUM_SUBCORES
_LANES = 16


def _tc_matvec_body(tbl_ref, w_ref, b_ref, o_ref):
    o_ref[...] = jnp.sum(tbl_ref[...] * w_ref[...], axis=0) + b_ref[0, 0]


def _tc_matvec(table, W, b, seq):
    vocab, emb = table.shape
    blkv = 8192
    grid = (vocab + blkv - 1) // blkv
    tT = table.T
    w_col = (W[0].astype(jnp.float32) / seq).reshape(emb, 1)
    b_scaled = jnp.reshape(b.astype(jnp.float32), (1, 1)) / seq
    return pl.pallas_call(
        _tc_matvec_body,
        grid=(grid,),
        in_specs=[
            pl.BlockSpec((emb, blkv), lambda i: (0, i)),
            pl.BlockSpec((emb, 1), lambda i: (0, 0)),
            pl.BlockSpec(memory_space=pltpu.SMEM),
        ],
        out_specs=pl.BlockSpec((blkv,), lambda i: (i,)),
        out_shape=jax.ShapeDtypeStruct((vocab,), jnp.float32),
    )(tT, w_col, b_scaled)


def _sc_gather_sum(x, t):
    seq, batch = x.shape
    bpw = batch // _NUM_WORKERS
    rows = 4
    nch = seq // rows
    nseg = bpw // 128

    mesh = plsc.VectorSubcoreMesh(
        core_axis_name="c", subcore_axis_name="s",
        num_cores=_NUM_CORES, num_subcores=_NUM_SUBCORES,
    )

    @functools.partial(
        pl.kernel,
        out_type=jax.ShapeDtypeStruct((batch,), jnp.float32),
        mesh=mesh,
        scratch_types=[
            pltpu.VMEM((rows, bpw), jnp.int32),
            pltpu.VMEM((rows, bpw), jnp.float32),
            pltpu.VMEM((bpw,), jnp.float32),
            pltpu.SemaphoreType.DMA,
        ],
    )
    def sc_kernel(x_hbm, t_hbm, out_hbm, idx_v, vals_v, acc_v, sem):
        wid = lax.axis_index("s") * _NUM_CORES + lax.axis_index("c")
        base = wid * bpw

        zero = jnp.zeros((_LANES,), jnp.float32)
        for c in range(bpw // _LANES):
            acc_v[pl.ds(c * _LANES, _LANES)] = zero

        @pl.loop(0, nch)
        def _chunk(i):
            s0 = i * rows
            pltpu.sync_copy(
                x_hbm.at[pl.ds(s0, rows), pl.ds(base, bpw)], idx_v
            )
            copies = []
            for r in range(rows):
                for c in range(nseg):
                    copies.append(pltpu.async_copy(
                        t_hbm.at[idx_v.at[r, pl.ds(c * 128, 128)]],
                        vals_v.at[r, pl.ds(c * 128, 128)],
                        sem,
                    ))
            for cp in copies:
                cp.wait()
            for c in range(bpw // _LANES):
                v = acc_v[pl.ds(c * _LANES, _LANES)]
                for r in range(rows):
                    v = v + vals_v[r, pl.ds(c * _LANES, _LANES)]
                acc_v[pl.ds(c * _LANES, _LANES)] = v

        pltpu.sync_copy(acc_v, out_hbm.at[pl.ds(base, bpw)])

    return sc_kernel(x, t)


def kernel(x, table, W, b):
    seq, _ = x.shape
    t = _tc_matvec(table, W, b, seq)
    return _sc_gather_sum(x, t)


# speedup vs baseline: 7.0477x; 1.1122x over previous
import functools

import jax
import jax.numpy as jnp
from jax import lax
from jax.experimental import pallas as pl
from jax.experimental.pallas import tpu as pltpu
from jax.experimental.pallas import tpu_sc as plsc

_NUM_CORES = 2
_NUM_SUBCORES = 16
_NUM_WORKERS = _NUM_CORES * _NUM_SUBCORES
_LANES = 16


def _tc_matvec_body(tbl_ref, w_ref, b_ref, o_ref):
    o_ref[...] = jnp.sum(tbl_ref[...] * w_ref[...], axis=0) + b_ref[0, 0]


def _tc_matvec(table, W, b, seq):
    vocab, emb = table.shape
    blkv = 8192
    grid = (vocab + blkv - 1) // blkv
    tT = table.T
    w_col = (W[0].astype(jnp.float32) / seq).reshape(emb, 1)
    b_scaled = jnp.reshape(b.astype(jnp.float32), (1, 1)) / seq
    return pl.pallas_call(
        _tc_matvec_body,
        grid=(grid,),
        in_specs=[
            pl.BlockSpec((emb, blkv), lambda i: (0, i)),
            pl.BlockSpec((emb, 1), lambda i: (0, 0)),
            pl.BlockSpec(memory_space=pltpu.SMEM),
        ],
        out_specs=pl.BlockSpec((blkv,), lambda i: (i,)),
        out_shape=jax.ShapeDtypeStruct((vocab,), jnp.float32),
    )(tT, w_col, b_scaled)


def _sc_gather_sum(x, t):
    seq, batch = x.shape
    bpw = batch // _NUM_WORKERS
    rows = 4
    nch = seq // rows
    nidx = rows * bpw
    nseg = nidx // 128

    mesh = plsc.VectorSubcoreMesh(
        core_axis_name="c", subcore_axis_name="s",
        num_cores=_NUM_CORES, num_subcores=_NUM_SUBCORES,
    )

    @functools.partial(
        pl.kernel,
        out_type=jax.ShapeDtypeStruct((batch,), jnp.float32),
        mesh=mesh,
        scratch_types=[
            pltpu.VMEM((2, rows * bpw), jnp.int32),
            pltpu.VMEM((2, rows * bpw), jnp.float32),
            pltpu.VMEM((bpw,), jnp.float32),
            pltpu.SemaphoreType.DMA,
            pltpu.SemaphoreType.DMA,
            pltpu.SemaphoreType.DMA,
            pltpu.SemaphoreType.DMA,
        ],
    )
    def sc_kernel(x_hbm, t_hbm, out_hbm, idx_v, vals_v, acc_v,
                  sx0, sx1, sg0, sg1):
        wid = lax.axis_index("s") * _NUM_CORES + lax.axis_index("c")
        base = wid * bpw
        sx = (sx0, sx1)
        sg = (sg0, sg1)

        def fire_x(i, buf):
            s0 = i * rows
            for r in range(rows):
                pltpu.async_copy(
                    x_hbm.at[s0 + r, pl.ds(base, bpw)],
                    idx_v.at[buf, pl.ds(r * bpw, bpw)], sx[buf])

        def wait_x(buf):
            for r in range(rows):
                pltpu.make_async_copy(
                    x_hbm.at[0, pl.ds(base, bpw)],
                    idx_v.at[buf, pl.ds(r * bpw, bpw)], sx[buf]).wait()

        def fire_g(buf):
            for k in range(nseg):
                pltpu.async_copy(
                    t_hbm.at[idx_v.at[buf, pl.ds(k * 128, 128)]],
                    vals_v.at[buf, pl.ds(k * 128, 128)], sg[buf])

        def wait_g(buf):
            for k in range(nseg):
                pltpu.make_async_copy(
                    t_hbm.at[idx_v.at[buf, pl.ds(k * 128, 128)]],
                    vals_v.at[buf, pl.ds(k * 128, 128)], sg[buf]).wait()

        def accumulate(buf):
            for l in range(bpw // _LANES):
                v = acc_v[pl.ds(l * _LANES, _LANES)]
                for r in range(rows):
                    v = v + vals_v[buf, pl.ds(r * bpw + l * _LANES, _LANES)]
                acc_v[pl.ds(l * _LANES, _LANES)] = v

        zero = jnp.zeros((_LANES,), jnp.float32)
        for c in range(bpw // _LANES):
            acc_v[pl.ds(c * _LANES, _LANES)] = zero

        fire_x(0, 0)

        @pl.loop(0, nch // 2)
        def _pair(j):
            a = 2 * j
            wait_x(0)
            fire_g(0)
            fire_x(a + 1, 1)
            wait_g(0)
            wait_x(1)
            fire_g(1)

            @pl.when(a + 2 < nch)
            def _prefetch():
                fire_x(a + 2, 0)

            accumulate(0)
            wait_g(1)
            accumulate(1)

        if nch % 2 == 1:
            wait_x(0)
            fire_g(0)
            wait_g(0)
            accumulate(0)

        pltpu.sync_copy(acc_v, out_hbm.at[pl.ds(base, bpw)])

    return sc_kernel(x, t)


def kernel(x, table, W, b):
    seq, _ = x.shape
    t = _tc_matvec(table, W, b, seq)
    return _sc_gather_sum(x, t)


# speedup vs baseline: 8.1467x; 1.1559x over previous
import functools

import jax
import jax.numpy as jnp
from jax import lax
from jax.experimental import pallas as pl
from jax.experimental.pallas import tpu as pltpu
from jax.experimental.pallas import tpu_sc as plsc

_NUM_CORES = 2
_NUM_SUBCORES = 16
_NUM_WORKERS = _NUM_CORES * _NUM_SUBCORES
_LANES = 16


def _tc_matvec_body(tbl_ref, w_ref, b_ref, o_ref):
    r = jnp.dot(w_ref[...], tbl_ref[...], preferred_element_type=jnp.float32)
    o_ref[...] = r.reshape(o_ref.shape) + b_ref[0, 0]


def _tc_matvec(table, W, b, seq):
    vocab, emb = table.shape
    blkv = 16384
    grid = (vocab + blkv - 1) // blkv
    tT = table.T
    w_row = (W.astype(jnp.float32) / seq)
    b_scaled = jnp.reshape(b.astype(jnp.float32), (1, 1)) / seq
    return pl.pallas_call(
        _tc_matvec_body,
        grid=(grid,),
        in_specs=[
            pl.BlockSpec((emb, blkv), lambda i: (0, i)),
            pl.BlockSpec((1, emb), lambda i: (0, 0)),
            pl.BlockSpec(memory_space=pltpu.SMEM),
        ],
        out_specs=pl.BlockSpec((blkv,), lambda i: (i,)),
        out_shape=jax.ShapeDtypeStruct((vocab,), jnp.float32),
    )(tT, w_row, b_scaled)


def _sc_gather_sum(x, t):
    seq, batch = x.shape
    bpw = batch // _NUM_WORKERS
    rows = 4
    nch = seq // rows
    nidx = rows * bpw
    nseg = nidx // 128

    mesh = plsc.VectorSubcoreMesh(
        core_axis_name="c", subcore_axis_name="s",
        num_cores=_NUM_CORES, num_subcores=_NUM_SUBCORES,
    )

    @functools.partial(
        pl.kernel,
        out_type=jax.ShapeDtypeStruct((batch,), jnp.float32),
        mesh=mesh,
        scratch_types=[
            pltpu.VMEM((2, rows * bpw), jnp.int32),
            pltpu.VMEM((2, rows * bpw), jnp.float32),
            pltpu.VMEM((bpw,), jnp.float32),
            pltpu.SemaphoreType.DMA,
            pltpu.SemaphoreType.DMA,
            pltpu.SemaphoreType.DMA,
            pltpu.SemaphoreType.DMA,
        ],
    )
    def sc_kernel(x_hbm, t_hbm, out_hbm, idx_v, vals_v, acc_v,
                  sx0, sx1, sg0, sg1):
        wid = lax.axis_index("s") * _NUM_CORES + lax.axis_index("c")
        base = wid * bpw
        sx = (sx0, sx1)
        sg = (sg0, sg1)

        def fire_x(i, buf):
            s0 = i * rows
            for r in range(rows):
                pltpu.async_copy(
                    x_hbm.at[s0 + r, pl.ds(base, bpw)],
                    idx_v.at[buf, pl.ds(r * bpw, bpw)], sx[buf])

        def wait_x(buf):
            for r in range(rows):
                pltpu.make_async_copy(
                    x_hbm.at[0, pl.ds(base, bpw)],
                    idx_v.at[buf, pl.ds(r * bpw, bpw)], sx[buf]).wait()

        def fire_g(buf):
            for k in range(nseg):
                pltpu.async_copy(
                    t_hbm.at[idx_v.at[buf, pl.ds(k * 128, 128)]],
                    vals_v.at[buf, pl.ds(k * 128, 128)], sg[buf])

        def wait_g(buf):
            for k in range(nseg):
                pltpu.make_async_copy(
                    t_hbm.at[idx_v.at[buf, pl.ds(k * 128, 128)]],
                    vals_v.at[buf, pl.ds(k * 128, 128)], sg[buf]).wait()

        def accumulate(buf):
            for l in range(bpw // _LANES):
                v = acc_v[pl.ds(l * _LANES, _LANES)]
                for r in range(rows):
                    v = v + vals_v[buf, pl.ds(r * bpw + l * _LANES, _LANES)]
                acc_v[pl.ds(l * _LANES, _LANES)] = v

        zero = jnp.zeros((_LANES,), jnp.float32)
        for c in range(bpw // _LANES):
            acc_v[pl.ds(c * _LANES, _LANES)] = zero

        fire_x(0, 0)

        @pl.loop(0, nch // 2)
        def _pair(j):
            a = 2 * j
            wait_x(0)
            fire_g(0)
            fire_x(a + 1, 1)
            wait_g(0)
            wait_x(1)
            fire_g(1)

            @pl.when(a + 2 < nch)
            def _prefetch():
                fire_x(a + 2, 0)

            accumulate(0)
            wait_g(1)
            accumulate(1)

        if nch % 2 == 1:
            wait_x(0)
            fire_g(0)
            wait_g(0)
            accumulate(0)

        pltpu.sync_copy(acc_v, out_hbm.at[pl.ds(base, bpw)])

    return sc_kernel(x, t)


def kernel(x, table, W, b):
    seq, _ = x.shape
    t = _tc_matvec(table, W, b, seq)
    return _sc_gather_sum(x, t)
